# baseline (speedup 1.0000x reference)
import functools

import jax
import jax.numpy as jnp
from jax import lax
from jax.experimental import pallas as pl
from jax.experimental.pallas import tpu as pltpu
from jax.experimental.pallas import tpu_sc as plsc

_NC = 2
_NS = 16
_NW = _NC * _NS
_LANES = 16
_K = 16
_QK = 80


def _node_mm_body(x_ref, wba_ref, bba_ref, wc_ref, bc_ref, wr_ref, br_ref,
                  rowtab_ref, coltab_ref, xr_ref):
    x = x_ref[...]
    rowtab_ref[...] = (
        jnp.dot(x, wba_ref[...], preferred_element_type=jnp.float32)
        + bba_ref[...]
    )
    coltab_ref[...] = (
        jnp.dot(x, wc_ref[...], preferred_element_type=jnp.float32)
        + bc_ref[...]
    )
    xr_ref[...] = (
        jnp.dot(x, wr_ref[...], preferred_element_type=jnp.float32)
        + br_ref[...]
    )


def _node_mm(x, wba, bba, wc, bc, wr, br):
    n, d = x.shape
    blk = 2000
    return pl.pallas_call(
        _node_mm_body,
        grid=(n // blk,),
        in_specs=[
            pl.BlockSpec((blk, d), lambda i: (i, 0)),
            pl.BlockSpec((d, 2 * d), lambda i: (0, 0)),
            pl.BlockSpec((1, 2 * d), lambda i: (0, 0)),
            pl.BlockSpec((d, d), lambda i: (0, 0)),
            pl.BlockSpec((1, d), lambda i: (0, 0)),
            pl.BlockSpec((d, d), lambda i: (0, 0)),
            pl.BlockSpec((1, d), lambda i: (0, 0)),
        ],
        out_specs=[
            pl.BlockSpec((blk, 2 * d), lambda i: (i, 0)),
            pl.BlockSpec((blk, d), lambda i: (i, 0)),
            pl.BlockSpec((blk, d), lambda i: (i, 0)),
        ],
        out_shape=[
            jax.ShapeDtypeStruct((n, 2 * d), jnp.float32),
            jax.ShapeDtypeStruct((n, d), jnp.float32),
            jax.ShapeDtypeStruct((n, d), jnp.float32),
        ],
    )(x, wba, bba, wc, bc, wr, br)


def _edge_mm_body(ea_ref, w_ref, b_ref, out_ref):
    out_ref[...] = (
        jnp.dot(ea_ref[...], w_ref[...], preferred_element_type=jnp.float32)
        + b_ref[...]
    )


def _edge_mm(edge_attr, w, b):
    e, d = edge_attr.shape
    blk = 4000
    return pl.pallas_call(
        _edge_mm_body,
        grid=(e // blk,),
        in_specs=[
            pl.BlockSpec((blk, d), lambda i: (i, 0)),
            pl.BlockSpec((d, d), lambda i: (0, 0)),
            pl.BlockSpec((1, d), lambda i: (0, 0)),
        ],
        out_specs=pl.BlockSpec((blk, d), lambda i: (i, 0)),
        out_shape=jax.ShapeDtypeStruct((e, d), jnp.float32),
    )(edge_attr, w, b)


def _combine_body(p_ref, xr_ref, out_ref):
    out_ref[...] = jnp.maximum(p_ref[0] + p_ref[1] + xr_ref[...], 0.0)


def _combine(partials, xr):
    n, d = xr.shape
    blk = 2000
    return pl.pallas_call(
        _combine_body,
        grid=(n // blk,),
        in_specs=[
            pl.BlockSpec((_NC, blk, d), lambda i: (0, i, 0)),
            pl.BlockSpec((blk, d), lambda i: (i, 0)),
        ],
        out_specs=pl.BlockSpec((blk, d), lambda i: (i, 0)),
        out_shape=jax.ShapeDtypeStruct((n, d), jnp.float32),
    )(partials, xr)


@functools.lru_cache(maxsize=None)
def _make_sc_gate(n, e, d):
    assert e % (_NW * _K) == 0 and n % _QK == 0 and d % _LANES == 0
    e_per_w = e // _NW
    n_chunks = e_per_w // _K
    n_stripes = n // _QK
    stripes_per_tile = (n_stripes + _NS - 1) // _NS
    mesh = plsc.VectorSubcoreMesh(core_axis_name="c", subcore_axis_name="s")

    @functools.partial(
        pl.kernel,
        out_type=jax.ShapeDtypeStruct((_NC, n, d), jnp.float32),
        mesh=mesh,
        scratch_types=[
            [pltpu.VMEM((2, _K), jnp.int32) for _ in range(4)],
            [pltpu.VMEM((_K, 2 * d), jnp.float32) for _ in range(2)],
            [pltpu.VMEM((_K, d), jnp.float32) for _ in range(2)],
            [pltpu.VMEM((_K, d), jnp.float32) for _ in range(2)],
            [pltpu.VMEM((_K, d), jnp.float32) for _ in range(2)],
            pltpu.VMEM((_QK, d), jnp.float32),
            pltpu.VMEM_SHARED((n, d), jnp.float32),
            [pltpu.SemaphoreType.DMA for _ in range(4)],
            [pltpu.SemaphoreType.DMA for _ in range(2)],
            [pltpu.SemaphoreType.DMA for _ in range(2)],
        ],
    )
    def sc_gate(rowtab, coltab, ee, idx2, out,
                ibuf, rbuf, cbuf, ebuf, mbuf, obuf, agg_sh, isem, gsem, ssem):
        c = lax.axis_index("c")
        s = lax.axis_index("s")
        wid = c * _NS + s

        def zrow(i, _):
            for j in range(d // _LANES):
                obuf[i, pl.ds(j * _LANES, _LANES)] = jnp.zeros(
                    (_LANES,), jnp.float32)
            return 0

        lax.fori_loop(0, _QK, zrow, 0)

        def zstripe(i, _):
            t = s + i * _NS

            @pl.when(t < n_stripes)
            def _():
                pltpu.sync_copy(obuf, agg_sh.at[pl.ds(t * _QK, _QK)])

            return 0

        lax.fori_loop(0, stripes_per_tile, zstripe, 0)
        plsc.subcore_barrier()

        base0 = wid * e_per_w

        def issue_idx(i, q):
            pltpu.async_copy(idx2.at[wid, i], ibuf[q], isem[q])

        def wait_idx(i, q):
            pltpu.make_async_copy(idx2.at[wid, i], ibuf[q], isem[q]).wait()

        def issue_gather(i, b, q):
            pltpu.async_copy(rowtab.at[ibuf[q].at[0]], rbuf[b], gsem[b])
            pltpu.async_copy(coltab.at[ibuf[q].at[1]], cbuf[b], gsem[b])
            pltpu.async_copy(ee.at[pl.ds(base0 + i * _K, _K)], ebuf[b],
                             gsem[b])

        def wait_gather(i, b, q):
            pltpu.make_async_copy(rowtab.at[ibuf[q].at[0]], rbuf[b],
                                  gsem[b]).wait()
            pltpu.make_async_copy(rowtab.at[ibuf[q].at[0]], rbuf[b],
                                  gsem[b]).wait()

        def step(i, b, q, qn, q2):
            @pl.when(i + 1 < n_chunks)
            def _():
                wait_idx(i + 1, qn)
                issue_gather(i + 1, 1 - b, qn)

            wait_gather(i, b, q)

            @pl.when(i >= 2)
            def _():
                pltpu.make_async_copy(mbuf[b], agg_sh.at[ibuf[q].at[1]],
                                      ssem[b]).wait()

            @pl.when(i + 2 < n_chunks)
            def _():
                issue_idx(i + 2, q2)

            @plsc.parallel_loop(0, _K, step=1, unroll=6)
            def _(iE):
                for j in range(d // _LANES):
                    ds_ = pl.ds(j * _LANES, _LANES)
                    z = (ebuf[b][iE, ds_] + rbuf[b][iE, ds_]
                         + cbuf[b][iE, ds_])
                    mbuf[b][iE, ds_] = (
                        rbuf[b][iE, pl.ds(d + j * _LANES, _LANES)]
                        / (1.0 + jnp.exp(z)))
            pltpu.async_copy(mbuf[b], agg_sh.at[ibuf[q].at[1]], ssem[b],
                             add=True)

        issue_idx(jnp.int32(0), 0)
        issue_idx(jnp.int32(1), 1)
        wait_idx(jnp.int32(0), 0)
        issue_gather(jnp.int32(0), 0, 0)

        def quad(t, _):
            i0 = 4 * t
            step(i0, 0, 0, 1, 2)
            step(i0 + 1, 1, 1, 2, 3)
            step(i0 + 2, 0, 2, 3, 0)
            step(i0 + 3, 1, 3, 0, 1)
            return 0

        lax.fori_loop(0, n_chunks // 4, quad, 0)
        for r in range(n_chunks % 4):
            i = n_chunks - (n_chunks % 4) + r
            step(jnp.int32(i), i % 2, i % 4, (i + 1) % 4, (i + 2) % 4)

        for b in range(2):
            pltpu.make_async_copy(mbuf[b], agg_sh.at[ibuf[0].at[1]],
                                  ssem[b]).wait()
        plsc.subcore_barrier()

        def ostripe(i, _):
            t = s + i * _NS

            @pl.when(t < n_stripes)
            def _():
                pltpu.sync_copy(agg_sh.at[pl.ds(t * _QK, _QK)], obuf)
                pltpu.sync_copy(obuf, out.at[c].at[pl.ds(t * _QK, _QK)])

            return 0

        lax.fori_loop(0, stripes_per_tile, ostripe, 0)

    return sc_gate


def kernel(x, edge_index, edge_attr,
           A_w, A_b, B_w, B_b, C_w, C_b, E_w, E_b, R_w, R_b):
    n, d = x.shape
    e = edge_attr.shape[0]
    e_per_w = e // _NW
    idx2 = jnp.stack(
        [edge_index[0].astype(jnp.int32).reshape(_NW, e_per_w // _K, _K),
         edge_index[1].astype(jnp.int32).reshape(_NW, e_per_w // _K, _K)],
        axis=2)

    wba = jnp.concatenate([-B_w, A_w], axis=1)
    bba = jnp.concatenate([-B_b, A_b])[None, :]
    rowtab, coltab, xr = _node_mm(x, wba, bba, -C_w, -C_b[None, :],
                                  R_w, R_b[None, :])
    ee = _edge_mm(edge_attr, -E_w, -E_b[None, :])

    partials = _make_sc_gate(n, e, d)(rowtab, coltab, ee, idx2)
    return _combine(partials, xr)


# speedup vs baseline: 1.7800x; 1.7800x over previous
import functools

import jax
import jax.numpy as jnp
from jax import lax
from jax.experimental import pallas as pl
from jax.experimental.pallas import tpu as pltpu
from jax.experimental.pallas import tpu_sc as plsc

_NC = 2
_NS = 16
_NW = _NC * _NS
_LANES = 16
_K = 16
_QK = 80


def _node_mm_body(x_ref, wba_ref, bba_ref, wc_ref, bc_ref, wr_ref, br_ref,
                  rowtab_ref, coltab_ref, xr_ref):
    x = x_ref[...]
    rowtab_ref[...] = (
        jnp.dot(x, wba_ref[...], preferred_element_type=jnp.float32)
        + bba_ref[...]
    )
    coltab_ref[...] = (
        jnp.dot(x, wc_ref[...], preferred_element_type=jnp.float32)
        + bc_ref[...]
    )
    xr_ref[...] = (
        jnp.dot(x, wr_ref[...], preferred_element_type=jnp.float32)
        + br_ref[...]
    )


def _node_mm(x, wba, bba, wc, bc, wr, br):
    n, d = x.shape
    blk = 2000
    return pl.pallas_call(
        _node_mm_body,
        grid=(n // blk,),
        in_specs=[
            pl.BlockSpec((blk, d), lambda i: (i, 0)),
            pl.BlockSpec((d, 2 * d), lambda i: (0, 0)),
            pl.BlockSpec((1, 2 * d), lambda i: (0, 0)),
            pl.BlockSpec((d, d), lambda i: (0, 0)),
            pl.BlockSpec((1, d), lambda i: (0, 0)),
            pl.BlockSpec((d, d), lambda i: (0, 0)),
            pl.BlockSpec((1, d), lambda i: (0, 0)),
        ],
        out_specs=[
            pl.BlockSpec((blk, 2 * d), lambda i: (i, 0)),
            pl.BlockSpec((blk, d), lambda i: (i, 0)),
            pl.BlockSpec((blk, d), lambda i: (i, 0)),
        ],
        out_shape=[
            jax.ShapeDtypeStruct((n, 2 * d), jnp.float32),
            jax.ShapeDtypeStruct((n, d), jnp.float32),
            jax.ShapeDtypeStruct((n, d), jnp.float32),
        ],
    )(x, wba, bba, wc, bc, wr, br)


def _edge_mm_body(ea_ref, w_ref, b_ref, out_ref):
    out_ref[...] = (
        jnp.dot(ea_ref[...], w_ref[...], preferred_element_type=jnp.float32)
        + b_ref[...]
    )


def _edge_mm(edge_attr, w, b):
    e, d = edge_attr.shape
    blk = 4000
    return pl.pallas_call(
        _edge_mm_body,
        grid=(e // blk,),
        in_specs=[
            pl.BlockSpec((blk, d), lambda i: (i, 0)),
            pl.BlockSpec((d, d), lambda i: (0, 0)),
            pl.BlockSpec((1, d), lambda i: (0, 0)),
        ],
        out_specs=pl.BlockSpec((blk, d), lambda i: (i, 0)),
        out_shape=jax.ShapeDtypeStruct((e, d), jnp.float32),
    )(edge_attr, w, b)


def _combine_body(p_ref, xr_ref, out_ref):
    out_ref[...] = jnp.maximum(p_ref[0] + p_ref[1] + xr_ref[...], 0.0)


def _combine(partials, xr):
    n, d = xr.shape
    blk = 2000
    return pl.pallas_call(
        _combine_body,
        grid=(n // blk,),
        in_specs=[
            pl.BlockSpec((_NC, blk, d), lambda i: (0, i, 0)),
            pl.BlockSpec((blk, d), lambda i: (i, 0)),
        ],
        out_specs=pl.BlockSpec((blk, d), lambda i: (i, 0)),
        out_shape=jax.ShapeDtypeStruct((n, d), jnp.float32),
    )(partials, xr)


@functools.lru_cache(maxsize=None)
def _make_sc_gate(n, e, d):
    assert e % (_NW * _K) == 0 and n % _QK == 0 and d % _LANES == 0
    e_per_w = e // _NW
    n_chunks = e_per_w // _K
    n_stripes = n // _QK
    stripes_per_tile = (n_stripes + _NS - 1) // _NS
    mesh = plsc.VectorSubcoreMesh(core_axis_name="c", subcore_axis_name="s")

    @functools.partial(
        pl.kernel,
        out_type=jax.ShapeDtypeStruct((_NC, n, d), jnp.float32),
        mesh=mesh,
        scratch_types=[
            [pltpu.VMEM((2, _K), jnp.int32) for _ in range(4)],
            [pltpu.VMEM((_K, 2 * d), jnp.float32) for _ in range(2)],
            [pltpu.VMEM((_K, d), jnp.float32) for _ in range(2)],
            [pltpu.VMEM((_K, d), jnp.float32) for _ in range(2)],
            [pltpu.VMEM((_K, d), jnp.float32) for _ in range(2)],
            pltpu.VMEM((_QK, d), jnp.float32),
            pltpu.VMEM_SHARED((n, d), jnp.float32),
            [pltpu.SemaphoreType.DMA for _ in range(4)],
            [pltpu.SemaphoreType.DMA for _ in range(2)],
            [pltpu.SemaphoreType.DMA for _ in range(2)],
        ],
    )
    def sc_gate(rowtab, coltab, ee, idx2, out,
                ibuf, rbuf, cbuf, ebuf, mbuf, obuf, agg_sh, isem, gsem, ssem):
        c = lax.axis_index("c")
        s = lax.axis_index("s")
        wid = c * _NS + s

        def zrow(i, _):
            for j in range(d // _LANES):
                obuf[i, pl.ds(j * _LANES, _LANES)] = jnp.zeros(
                    (_LANES,), jnp.float32)
            return 0

        lax.fori_loop(0, _QK, zrow, 0)

        def zstripe(i, _):
            t = s + i * _NS

            @pl.when(t < n_stripes)
            def _():
                pltpu.sync_copy(obuf, agg_sh.at[pl.ds(t * _QK, _QK)])

            return 0

        lax.fori_loop(0, stripes_per_tile, zstripe, 0)
        plsc.subcore_barrier()

        base0 = wid * e_per_w

        def issue_idx(i, q):
            pltpu.async_copy(idx2.at[wid, i], ibuf[q], isem[q])

        def wait_idx(i, q):
            pltpu.make_async_copy(idx2.at[wid, i], ibuf[q], isem[q]).wait()

        def issue_gather(i, b, q):
            pltpu.async_copy(rowtab.at[ibuf[q].at[0]], rbuf[b], gsem[b])
            pltpu.async_copy(coltab.at[ibuf[q].at[1]], cbuf[b], gsem[b])
            pltpu.async_copy(ee.at[pl.ds(base0 + i * _K, _K)], ebuf[b],
                             gsem[b])

        def wait_gather(i, b, q):
            pltpu.make_async_copy(rowtab.at[ibuf[q].at[0]], rbuf[b],
                                  gsem[b]).wait()
            pltpu.make_async_copy(rowtab.at[ibuf[q].at[0]], rbuf[b],
                                  gsem[b]).wait()

        def step(i, b, q, qn, q2):
            @pl.when(i + 1 < n_chunks)
            def _():
                wait_idx(i + 1, qn)
                issue_gather(i + 1, 1 - b, qn)

            wait_gather(i, b, q)

            @pl.when(i >= 2)
            def _():
                pltpu.make_async_copy(mbuf[b], agg_sh.at[ibuf[q].at[1]],
                                      ssem[b]).wait()

            @pl.when(i + 2 < n_chunks)
            def _():
                issue_idx(i + 2, q2)

            @plsc.parallel_loop(0, _K, step=1, unroll=4)
            def _(iE):
                for j in range(d // _LANES):
                    ds_ = pl.ds(j * _LANES, _LANES)
                    z = (ebuf[b][iE, ds_] + rbuf[b][iE, ds_]
                         + cbuf[b][iE, ds_])
                    mbuf[b][iE, ds_] = (
                        rbuf[b][iE, pl.ds(d + j * _LANES, _LANES)]
                        / (1.0 + jnp.exp(z)))
            pltpu.async_copy(mbuf[b], agg_sh.at[ibuf[q].at[1]], ssem[b],
                             add=True)

        issue_idx(jnp.int32(0), 0)
        issue_idx(jnp.int32(1), 1)
        wait_idx(jnp.int32(0), 0)
        issue_gather(jnp.int32(0), 0, 0)

        def quad(t, _):
            i0 = 4 * t
            step(i0, 0, 0, 1, 2)
            step(i0 + 1, 1, 1, 2, 3)
            step(i0 + 2, 0, 2, 3, 0)
            step(i0 + 3, 1, 3, 0, 1)
            return 0

        lax.fori_loop(0, n_chunks // 4, quad, 0)
        for r in range(n_chunks % 4):
            i = n_chunks - (n_chunks % 4) + r
            step(jnp.int32(i), i % 2, i % 4, (i + 1) % 4, (i + 2) % 4)

        for b in range(2):
            pltpu.make_async_copy(mbuf[b], agg_sh.at[ibuf[0].at[1]],
                                  ssem[b]).wait()
        plsc.subcore_barrier()

        def ostripe(i, _):
            t = s + i * _NS

            @pl.when(t < n_stripes)
            def _():
                pltpu.sync_copy(agg_sh.at[pl.ds(t * _QK, _QK)], obuf)
                pltpu.sync_copy(obuf, out.at[c].at[pl.ds(t * _QK, _QK)])

            return 0

        lax.fori_loop(0, stripes_per_tile, ostripe, 0)

    return sc_gate


def kernel(x, edge_index, edge_attr,
           A_w, A_b, B_w, B_b, C_w, C_b, E_w, E_b, R_w, R_b):
    n, d = x.shape
    e = edge_attr.shape[0]
    e_per_w = e // _NW
    idx2 = jnp.stack(
        [edge_index[0].astype(jnp.int32).reshape(_NW, e_per_w // _K, _K),
         edge_index[1].astype(jnp.int32).reshape(_NW, e_per_w // _K, _K)],
        axis=2)

    wba = jnp.concatenate([-B_w, A_w], axis=1)
    bba = jnp.concatenate([-B_b, A_b])[None, :]
    rowtab, coltab, xr = _node_mm(x, wba, bba, -C_w, -C_b[None, :],
                                  R_w, R_b[None, :])
    ee = _edge_mm(edge_attr, -E_w, -E_b[None, :])

    partials = _make_sc_gate(n, e, d)(rowtab, coltab, ee, idx2)
    return _combine(partials, xr)
